# Initial kernel scaffold; baseline (speedup 1.0000x reference)
import jax
import jax.numpy as jnp
from jax.experimental import pallas as pl


def kernel(x, edge_index, W1, b1, W2, b2):
    raise NotImplementedError("write your pallas kernel here")


# speedup vs baseline: 29.5141x; 29.5141x over previous
import functools

import jax
import jax.numpy as jnp
from jax import lax
from jax.experimental import pallas as pl
from jax.experimental.pallas import tpu as pltpu
from jax.experimental.pallas import tpu_sc as plsc

N_NODES = 100000
HIDDEN = 16
N_EDGES = 3200000
LANE = 128
EROWS = N_EDGES // LANE
NPAD = 100352
RPAD = NPAD // LANE
NUM_CORES = 2
NUM_SUBCORES = 16
NUM_WORKERS = NUM_CORES * NUM_SUBCORES
OUT_SLICE = NPAD // NUM_SUBCORES

ROWS_BASE = EROWS // NUM_WORKERS
ROWS_EXTRA = EROWS % NUM_WORKERS


def _worker_range(w):
    lo = w * ROWS_BASE + jnp.minimum(w, ROWS_EXTRA)
    hi = lo + ROWS_BASE + jnp.where(w < ROWS_EXTRA, 1, 0)
    return lo, hi


def _mesh():
    return plsc.VectorSubcoreMesh(core_axis_name="c", subcore_axis_name="s")


@functools.partial(
    pl.kernel,
    out_type=jax.ShapeDtypeStruct((NUM_CORES, NPAD), jnp.float32),
    mesh=_mesh(),
    scratch_types=[
        pltpu.VMEM((1, LANE), jnp.int32),
        pltpu.VMEM((LANE,), jnp.float32),
        pltpu.VMEM_SHARED((NPAD,), jnp.float32),
    ],
)
def _sc_count(edge_hbm, zeros_hbm, out_hbm, idx_v, ones_v, acc_sh):
    cid = lax.axis_index("c")
    sid = lax.axis_index("s")
    w = sid * NUM_CORES + cid

    for i in range(LANE // 16):
        ones_v[pl.ds(i * 16, 16)] = jnp.full((16,), 1.0, jnp.float32)

    @pl.when(sid == 0)
    def _():
        pltpu.sync_copy(zeros_hbm, acc_sh)

    plsc.subcore_barrier()

    lo, hi = _worker_range(w)

    def body(r, carry):
        pltpu.sync_copy(edge_hbm.at[1, r], idx_v.at[0])
        pltpu.sync_copy(ones_v, acc_sh.at[idx_v.at[0]], add=True)
        return carry

    lax.fori_loop(lo, hi, body, 0)

    plsc.subcore_barrier()
    pltpu.sync_copy(
        acc_sh.at[pl.ds(sid * OUT_SLICE, OUT_SLICE)],
        out_hbm.at[cid, pl.ds(sid * OUT_SLICE, OUT_SLICE)],
    )


@functools.partial(
    pl.kernel,
    out_type=jax.ShapeDtypeStruct((NUM_CORES, NPAD), jnp.float32),
    mesh=_mesh(),
    scratch_types=[
        pltpu.VMEM((2, LANE), jnp.int32),
        pltpu.VMEM((LANE,), jnp.float32),
        pltpu.VMEM_SHARED((NPAD,), jnp.float32),
        pltpu.SemaphoreType.DMA,
    ],
)
def _sc_seg(edge_hbm, zeros_hbm, tab_hbm, out_hbm, idx_v, val_v, acc_sh, sem):
    cid = lax.axis_index("c")
    sid = lax.axis_index("s")
    w = sid * NUM_CORES + cid

    @pl.when(sid == 0)
    def _():
        pltpu.sync_copy(zeros_hbm, acc_sh)

    plsc.subcore_barrier()

    lo, hi = _worker_range(w)

    def body(r, carry):
        pltpu.sync_copy(edge_hbm.at[0, r], idx_v.at[0])
        pltpu.sync_copy(edge_hbm.at[1, r], idx_v.at[1])
        pltpu.async_copy(tab_hbm.at[idx_v.at[0]], val_v, sem).wait()
        pltpu.sync_copy(val_v, acc_sh.at[idx_v.at[1]], add=True)
        return carry

    lax.fori_loop(lo, hi, body, 0)

    plsc.subcore_barrier()
    pltpu.sync_copy(
        acc_sh.at[pl.ds(sid * OUT_SLICE, OUT_SLICE)],
        out_hbm.at[cid, pl.ds(sid * OUT_SLICE, OUT_SLICE)],
    )


def _d1_body(p_ref, x_ref, dinv_ref, u_ref):
    deg = p_ref[0] + p_ref[1] + 1.0
    dinv = lax.rsqrt(deg)
    dinv_ref[...] = dinv
    u_ref[...] = dinv * x_ref[...]


_d1 = pl.pallas_call(
    _d1_body,
    out_shape=(
        jax.ShapeDtypeStruct((RPAD, LANE), jnp.float32),
        jax.ShapeDtypeStruct((RPAD, LANE), jnp.float32),
    ),
)


def _d2_body(q_ref, dinv_ref, u_ref, w1_ref, b1_ref, w2_ref, v_ref):
    dinv = dinv_ref[...]
    s1 = dinv * (q_ref[0] + q_ref[1] + u_ref[...])
    t = jnp.zeros_like(s1)
    for k in range(HIDDEN):
        t = t + jnp.maximum(s1 * w1_ref[0, k] + b1_ref[0, k], 0.0) * w2_ref[0, k]
    v_ref[...] = dinv * t


_d2 = pl.pallas_call(
    _d2_body,
    in_specs=[
        pl.BlockSpec(memory_space=pltpu.VMEM),
        pl.BlockSpec(memory_space=pltpu.VMEM),
        pl.BlockSpec(memory_space=pltpu.VMEM),
        pl.BlockSpec(memory_space=pltpu.SMEM),
        pl.BlockSpec(memory_space=pltpu.SMEM),
        pl.BlockSpec(memory_space=pltpu.SMEM),
    ],
    out_shape=jax.ShapeDtypeStruct((RPAD, LANE), jnp.float32),
)


def _d3_body(r_ref, dinv_ref, v_ref, b2_ref, o_ref):
    o_ref[...] = dinv_ref[...] * (r_ref[0] + r_ref[1] + v_ref[...]) + b2_ref[0]


_d3 = pl.pallas_call(
    _d3_body,
    in_specs=[
        pl.BlockSpec(memory_space=pltpu.VMEM),
        pl.BlockSpec(memory_space=pltpu.VMEM),
        pl.BlockSpec(memory_space=pltpu.VMEM),
        pl.BlockSpec(memory_space=pltpu.SMEM),
    ],
    out_shape=jax.ShapeDtypeStruct((RPAD, LANE), jnp.float32),
)


def kernel(x, edge_index, W1, b1, W2, b2):
    ei = edge_index.astype(jnp.int32).reshape(2, EROWS, LANE)
    xpad = jnp.pad(x[:, 0], (0, NPAD - N_NODES)).reshape(RPAD, LANE)
    zeros = jnp.zeros((NPAD,), jnp.float32)

    p = _sc_count(ei, zeros)
    dinv, u = _d1(p.reshape(NUM_CORES, RPAD, LANE), xpad)

    g1 = _sc_seg(ei, zeros, u.reshape(NPAD))
    v = _d2(
        g1.reshape(NUM_CORES, RPAD, LANE),
        dinv,
        u,
        W1.reshape(1, HIDDEN),
        b1.reshape(1, HIDDEN),
        W2.reshape(1, HIDDEN),
    )

    g2 = _sc_seg(ei, zeros, v.reshape(NPAD))
    out = _d3(g2.reshape(NUM_CORES, RPAD, LANE), dinv, v, b2)

    return out.reshape(NPAD)[:N_NODES].reshape(N_NODES, 1)


# speedup vs baseline: 117.1580x; 3.9696x over previous
import functools

import jax
import jax.numpy as jnp
from jax import lax
from jax.experimental import pallas as pl
from jax.experimental.pallas import tpu as pltpu
from jax.experimental.pallas import tpu_sc as plsc

N_NODES = 100000
HIDDEN = 16
N_EDGES = 3200000
LANE = 128
EROWS = N_EDGES // LANE
NPAD = 100352
RPAD = NPAD // LANE
NUM_CORES = 2
NUM_SUBCORES = 16
NUM_WORKERS = NUM_CORES * NUM_SUBCORES
OUT_SLICE = NPAD // NUM_SUBCORES

CH = 8
NBLK = EROWS // CH
BLK_BASE = NBLK // NUM_WORKERS
BLK_EXTRA = NBLK % NUM_WORKERS


def _blk_range(w):
    lo = w * BLK_BASE + jnp.minimum(w, BLK_EXTRA)
    hi = lo + BLK_BASE + jnp.where(w < BLK_EXTRA, 1, 0)
    return lo, hi


def _mesh():
    return plsc.VectorSubcoreMesh(core_axis_name="c", subcore_axis_name="s")


@functools.partial(
    pl.kernel,
    out_type=jax.ShapeDtypeStruct((NUM_CORES, NPAD), jnp.float32),
    mesh=_mesh(),
    scratch_types=[
        pltpu.VMEM((CH, LANE), jnp.int32),
        pltpu.VMEM((LANE,), jnp.float32),
        pltpu.VMEM_SHARED((NPAD,), jnp.float32),
        pltpu.SemaphoreType.DMA,
    ],
)
def _sc_count(edge_hbm, zeros_hbm, out_hbm, idx_v, ones_v, acc_sh, ssem):
    cid = lax.axis_index("c")
    sid = lax.axis_index("s")
    w = sid * NUM_CORES + cid

    for i in range(LANE // 16):
        ones_v[pl.ds(i * 16, 16)] = jnp.full((16,), 1.0, jnp.float32)

    @pl.when(sid == 0)
    def _():
        pltpu.sync_copy(zeros_hbm, acc_sh)

    plsc.subcore_barrier()

    lo, hi = _blk_range(w)

    def body(b, carry):
        pltpu.sync_copy(edge_hbm.at[1, pl.ds(b * CH, CH)], idx_v)
        hs = [
            pltpu.async_copy(ones_v, acc_sh.at[idx_v.at[j]], ssem, add=True)
            for j in range(CH)
        ]
        for h in hs:
            h.wait()
        return carry

    lax.fori_loop(lo, hi, body, 0)

    plsc.subcore_barrier()
    pltpu.sync_copy(
        acc_sh.at[pl.ds(sid * OUT_SLICE, OUT_SLICE)],
        out_hbm.at[cid, pl.ds(sid * OUT_SLICE, OUT_SLICE)],
    )


@functools.partial(
    pl.kernel,
    out_type=jax.ShapeDtypeStruct((NUM_CORES, NPAD), jnp.float32),
    mesh=_mesh(),
    scratch_types=[
        pltpu.VMEM((CH, LANE), jnp.int32),
        pltpu.VMEM((CH, LANE), jnp.int32),
        pltpu.VMEM((CH, LANE), jnp.float32),
        pltpu.VMEM_SHARED((NPAD,), jnp.float32),
        pltpu.SemaphoreType.DMA,
        pltpu.SemaphoreType.DMA,
    ],
)
def _sc_seg(edge_hbm, zeros_hbm, tab_hbm, out_hbm, sidx_v, didx_v, val_v, acc_sh, gsem, ssem):
    cid = lax.axis_index("c")
    sid = lax.axis_index("s")
    w = sid * NUM_CORES + cid

    @pl.when(sid == 0)
    def _():
        pltpu.sync_copy(zeros_hbm, acc_sh)

    plsc.subcore_barrier()

    lo, hi = _blk_range(w)

    def body(b, carry):
        rb = b * CH
        pltpu.sync_copy(edge_hbm.at[0, pl.ds(rb, CH)], sidx_v)
        pltpu.sync_copy(edge_hbm.at[1, pl.ds(rb, CH)], didx_v)
        gs = [
            pltpu.async_copy(tab_hbm.at[sidx_v.at[j]], val_v.at[j], gsem)
            for j in range(CH)
        ]
        for h in gs:
            h.wait()
        ss = [
            pltpu.async_copy(val_v.at[j], acc_sh.at[didx_v.at[j]], ssem, add=True)
            for j in range(CH)
        ]
        for h in ss:
            h.wait()
        return carry

    lax.fori_loop(lo, hi, body, 0)

    plsc.subcore_barrier()
    pltpu.sync_copy(
        acc_sh.at[pl.ds(sid * OUT_SLICE, OUT_SLICE)],
        out_hbm.at[cid, pl.ds(sid * OUT_SLICE, OUT_SLICE)],
    )


def _d1_body(p_ref, x_ref, dinv_ref, u_ref):
    deg = p_ref[0] + p_ref[1] + 1.0
    dinv = lax.rsqrt(deg)
    dinv_ref[...] = dinv
    u_ref[...] = dinv * x_ref[...]


_d1 = pl.pallas_call(
    _d1_body,
    out_shape=(
        jax.ShapeDtypeStruct((RPAD, LANE), jnp.float32),
        jax.ShapeDtypeStruct((RPAD, LANE), jnp.float32),
    ),
)


def _d2_body(q_ref, dinv_ref, u_ref, w1_ref, b1_ref, w2_ref, v_ref):
    dinv = dinv_ref[...]
    s1 = dinv * (q_ref[0] + q_ref[1] + u_ref[...])
    t = jnp.zeros_like(s1)
    for k in range(HIDDEN):
        t = t + jnp.maximum(s1 * w1_ref[0, k] + b1_ref[0, k], 0.0) * w2_ref[0, k]
    v_ref[...] = dinv * t


_d2 = pl.pallas_call(
    _d2_body,
    in_specs=[
        pl.BlockSpec(memory_space=pltpu.VMEM),
        pl.BlockSpec(memory_space=pltpu.VMEM),
        pl.BlockSpec(memory_space=pltpu.VMEM),
        pl.BlockSpec(memory_space=pltpu.SMEM),
        pl.BlockSpec(memory_space=pltpu.SMEM),
        pl.BlockSpec(memory_space=pltpu.SMEM),
    ],
    out_shape=jax.ShapeDtypeStruct((RPAD, LANE), jnp.float32),
)


def _d3_body(r_ref, dinv_ref, v_ref, b2_ref, o_ref):
    o_ref[...] = dinv_ref[...] * (r_ref[0] + r_ref[1] + v_ref[...]) + b2_ref[0]


_d3 = pl.pallas_call(
    _d3_body,
    in_specs=[
        pl.BlockSpec(memory_space=pltpu.VMEM),
        pl.BlockSpec(memory_space=pltpu.VMEM),
        pl.BlockSpec(memory_space=pltpu.VMEM),
        pl.BlockSpec(memory_space=pltpu.SMEM),
    ],
    out_shape=jax.ShapeDtypeStruct((RPAD, LANE), jnp.float32),
)


def kernel(x, edge_index, W1, b1, W2, b2):
    ei = edge_index.astype(jnp.int32).reshape(2, EROWS, LANE)
    xpad = jnp.pad(x[:, 0], (0, NPAD - N_NODES)).reshape(RPAD, LANE)
    zeros = jnp.zeros((NPAD,), jnp.float32)

    p = _sc_count(ei, zeros)
    dinv, u = _d1(p.reshape(NUM_CORES, RPAD, LANE), xpad)

    g1 = _sc_seg(ei, zeros, u.reshape(NPAD))
    v = _d2(
        g1.reshape(NUM_CORES, RPAD, LANE),
        dinv,
        u,
        W1.reshape(1, HIDDEN),
        b1.reshape(1, HIDDEN),
        W2.reshape(1, HIDDEN),
    )

    g2 = _sc_seg(ei, zeros, v.reshape(NPAD))
    out = _d3(g2.reshape(NUM_CORES, RPAD, LANE), dinv, v, b2)

    return out.reshape(NPAD)[:N_NODES].reshape(N_NODES, 1)


# speedup vs baseline: 239.8708x; 2.0474x over previous
import functools

import jax
import jax.numpy as jnp
from jax import lax
from jax.experimental import pallas as pl
from jax.experimental.pallas import tpu as pltpu
from jax.experimental.pallas import tpu_sc as plsc

N_NODES = 100000
HIDDEN = 16
N_EDGES = 3200000
LANE = 128
EROWS = N_EDGES // LANE
NPAD = 100352
RPAD = NPAD // LANE
NUM_CORES = 2
NUM_SUBCORES = 16
NUM_WORKERS = NUM_CORES * NUM_SUBCORES
OUT_SLICE = NPAD // NUM_SUBCORES

CHK = 16
CPW = 49
EROWSP = NUM_WORKERS * CPW * CHK

CH = 8
NBLK = EROWSP // CH
BLK_BASE = NBLK // NUM_WORKERS
BLK_EXTRA = NBLK % NUM_WORKERS


def _blk_range(w):
    lo = w * BLK_BASE + jnp.minimum(w, BLK_EXTRA)
    hi = lo + BLK_BASE + jnp.where(w < BLK_EXTRA, 1, 0)
    return lo, hi


def _mesh():
    return plsc.VectorSubcoreMesh(core_axis_name="c", subcore_axis_name="s")


@functools.partial(
    pl.kernel,
    out_type=jax.ShapeDtypeStruct((NUM_CORES, NPAD), jnp.float32),
    mesh=_mesh(),
    scratch_types=[
        pltpu.VMEM((CH, LANE), jnp.int32),
        pltpu.VMEM((LANE,), jnp.float32),
        pltpu.VMEM_SHARED((NPAD,), jnp.float32),
        pltpu.SemaphoreType.DMA,
    ],
)
def _sc_count(edge_hbm, zeros_hbm, out_hbm, idx_v, ones_v, acc_sh, ssem):
    cid = lax.axis_index("c")
    sid = lax.axis_index("s")
    w = sid * NUM_CORES + cid

    for i in range(LANE // 16):
        ones_v[pl.ds(i * 16, 16)] = jnp.full((16,), 1.0, jnp.float32)

    @pl.when(sid == 0)
    def _():
        pltpu.sync_copy(zeros_hbm, acc_sh)

    plsc.subcore_barrier()

    lo, hi = _blk_range(w)

    def body(b, carry):
        pltpu.sync_copy(edge_hbm.at[1, pl.ds(b * CH, CH)], idx_v)
        hs = [
            pltpu.async_copy(ones_v, acc_sh.at[idx_v.at[j]], ssem, add=True)
            for j in range(CH)
        ]
        for h in hs:
            h.wait()
        return carry

    lax.fori_loop(lo, hi, body, 0)

    plsc.subcore_barrier()
    pltpu.sync_copy(
        acc_sh.at[pl.ds(sid * OUT_SLICE, OUT_SLICE)],
        out_hbm.at[cid, pl.ds(sid * OUT_SLICE, OUT_SLICE)],
    )


@functools.partial(
    pl.kernel,
    out_type=jax.ShapeDtypeStruct((NUM_CORES, NPAD), jnp.float32),
    mesh=_mesh(),
    scratch_types=[
        pltpu.VMEM((NPAD,), jnp.float32),
        pltpu.VMEM((CHK, LANE), jnp.int32),
        pltpu.VMEM((CHK, LANE), jnp.int32),
        pltpu.VMEM((CHK, LANE), jnp.float32),
        pltpu.VMEM_SHARED((NPAD,), jnp.float32),
        pltpu.SemaphoreType.DMA,
    ],
    compiler_params=pltpu.CompilerParams(needs_layout_passes=False),
)
def _sc_seg(edge_hbm, zeros_hbm, tab_hbm, out_hbm, tab_v, sidx_v, didx_v, val_v, acc_sh, ssem):
    cid = lax.axis_index("c")
    sid = lax.axis_index("s")
    w = sid * NUM_CORES + cid

    @pl.when(sid == 0)
    def _():
        pltpu.sync_copy(zeros_hbm, acc_sh)

    pltpu.sync_copy(tab_hbm, tab_v)
    plsc.subcore_barrier()

    base = w * (CPW * CHK)

    def chunk(c, carry):
        r0 = base + c * CHK
        pltpu.sync_copy(edge_hbm.at[0, pl.ds(r0, CHK)], sidx_v)
        pltpu.sync_copy(edge_hbm.at[1, pl.ds(r0, CHK)], didx_v)
        hs = []
        for j in range(CHK):
            for i in range(LANE // 16):
                idx16 = sidx_v[j, pl.ds(i * 16, 16)]
                val_v[j, pl.ds(i * 16, 16)] = plsc.load_gather(tab_v, [idx16])
            hs.append(
                pltpu.async_copy(val_v.at[j], acc_sh.at[didx_v.at[j]], ssem, add=True)
            )
        for h in hs:
            h.wait()
        return carry

    lax.fori_loop(0, CPW, chunk, 0)

    plsc.subcore_barrier()
    pltpu.sync_copy(
        acc_sh.at[pl.ds(sid * OUT_SLICE, OUT_SLICE)],
        out_hbm.at[cid, pl.ds(sid * OUT_SLICE, OUT_SLICE)],
    )


def _d1_body(p_ref, x_ref, dinv_ref, u_ref):
    deg = p_ref[0] + p_ref[1] + 1.0
    dinv = lax.rsqrt(deg)
    dinv_ref[...] = dinv
    u_ref[...] = dinv * x_ref[...]


_d1 = pl.pallas_call(
    _d1_body,
    out_shape=(
        jax.ShapeDtypeStruct((RPAD, LANE), jnp.float32),
        jax.ShapeDtypeStruct((RPAD, LANE), jnp.float32),
    ),
)


def _d2_body(q_ref, dinv_ref, u_ref, w1_ref, b1_ref, w2_ref, v_ref):
    dinv = dinv_ref[...]
    s1 = dinv * (q_ref[0] + q_ref[1] + u_ref[...])
    t = jnp.zeros_like(s1)
    for k in range(HIDDEN):
        t = t + jnp.maximum(s1 * w1_ref[0, k] + b1_ref[0, k], 0.0) * w2_ref[0, k]
    v_ref[...] = dinv * t


_d2 = pl.pallas_call(
    _d2_body,
    in_specs=[
        pl.BlockSpec(memory_space=pltpu.VMEM),
        pl.BlockSpec(memory_space=pltpu.VMEM),
        pl.BlockSpec(memory_space=pltpu.VMEM),
        pl.BlockSpec(memory_space=pltpu.SMEM),
        pl.BlockSpec(memory_space=pltpu.SMEM),
        pl.BlockSpec(memory_space=pltpu.SMEM),
    ],
    out_shape=jax.ShapeDtypeStruct((RPAD, LANE), jnp.float32),
)


def _d3_body(r_ref, dinv_ref, v_ref, b2_ref, o_ref):
    o_ref[...] = dinv_ref[...] * (r_ref[0] + r_ref[1] + v_ref[...]) + b2_ref[0]


_d3 = pl.pallas_call(
    _d3_body,
    in_specs=[
        pl.BlockSpec(memory_space=pltpu.VMEM),
        pl.BlockSpec(memory_space=pltpu.VMEM),
        pl.BlockSpec(memory_space=pltpu.VMEM),
        pl.BlockSpec(memory_space=pltpu.SMEM),
    ],
    out_shape=jax.ShapeDtypeStruct((RPAD, LANE), jnp.float32),
)


def kernel(x, edge_index, W1, b1, W2, b2):
    ei = jnp.pad(
        edge_index.astype(jnp.int32),
        ((0, 0), (0, EROWSP * LANE - N_EDGES)),
        constant_values=N_NODES,
    ).reshape(2, EROWSP, LANE)
    xpad = jnp.pad(x[:, 0], (0, NPAD - N_NODES)).reshape(RPAD, LANE)
    zeros = jnp.zeros((NPAD,), jnp.float32)

    p = _sc_count(ei, zeros)
    dinv, u = _d1(p.reshape(NUM_CORES, RPAD, LANE), xpad)

    g1 = _sc_seg(ei, zeros, u.reshape(NPAD))
    v = _d2(
        g1.reshape(NUM_CORES, RPAD, LANE),
        dinv,
        u,
        W1.reshape(1, HIDDEN),
        b1.reshape(1, HIDDEN),
        W2.reshape(1, HIDDEN),
    )

    g2 = _sc_seg(ei, zeros, v.reshape(NPAD))
    out = _d3(g2.reshape(NUM_CORES, RPAD, LANE), dinv, v, b2)

    return out.reshape(NPAD)[:N_NODES].reshape(N_NODES, 1)


# speedup vs baseline: 319.5861x; 1.3323x over previous
import functools

import jax
import jax.numpy as jnp
from jax import lax
from jax.experimental import pallas as pl
from jax.experimental.pallas import tpu as pltpu
from jax.experimental.pallas import tpu_sc as plsc

N_NODES = 100000
HIDDEN = 16
N_EDGES = 3200000
LANE = 128
EROWS = N_EDGES // LANE
NPAD = 100352
RPAD = NPAD // LANE
NUM_CORES = 2
NUM_SUBCORES = 16
NUM_WORKERS = NUM_CORES * NUM_SUBCORES
OUT_SLICE = NPAD // NUM_SUBCORES

CH = 8
BPW = 98
EROWSP = NUM_WORKERS * BPW * CH
HALF = BPW // 2


def _mesh():
    return plsc.VectorSubcoreMesh(core_axis_name="c", subcore_axis_name="s")


@functools.partial(
    pl.kernel,
    out_type=jax.ShapeDtypeStruct((NUM_CORES, NPAD), jnp.float32),
    mesh=_mesh(),
    scratch_types=[
        pltpu.VMEM((CH, LANE), jnp.int32),
        pltpu.VMEM((CH, LANE), jnp.int32),
        pltpu.VMEM((LANE,), jnp.float32),
        pltpu.VMEM_SHARED((NPAD,), jnp.float32),
        pltpu.SemaphoreType.DMA,
        pltpu.SemaphoreType.DMA,
        pltpu.SemaphoreType.DMA,
        pltpu.SemaphoreType.DMA,
    ],
)
def _sc_count(edge_hbm, zeros_hbm, out_hbm, didx_a, didx_b, ones_v, acc_sh,
              isem_a, isem_b, ssem_a, ssem_b):
    cid = lax.axis_index("c")
    sid = lax.axis_index("s")
    w = sid * NUM_CORES + cid

    for i in range(LANE // 16):
        ones_v[pl.ds(i * 16, 16)] = jnp.full((16,), 1.0, jnp.float32)

    @pl.when(sid == 0)
    def _():
        pltpu.sync_copy(zeros_hbm, acc_sh)

    plsc.subcore_barrier()

    lo = w * BPW
    pltpu.async_copy(edge_hbm.at[1, pl.ds(lo * CH, CH)], didx_a, isem_a)
    pltpu.async_copy(edge_hbm.at[1, pl.ds((lo + 1) * CH, CH)], didx_b, isem_b)

    def body(i, carry):
        b_a = lo + 2 * i
        b_b = b_a + 1
        pltpu.make_async_copy(
            edge_hbm.at[1, pl.ds(b_a * CH, CH)], didx_a, isem_a
        ).wait()
        hs_a = [
            pltpu.async_copy(ones_v, acc_sh.at[didx_a.at[j]], ssem_a, add=True)
            for j in range(CH)
        ]
        pltpu.make_async_copy(
            edge_hbm.at[1, pl.ds(b_b * CH, CH)], didx_b, isem_b
        ).wait()
        hs_b = [
            pltpu.async_copy(ones_v, acc_sh.at[didx_b.at[j]], ssem_b, add=True)
            for j in range(CH)
        ]
        for h in hs_a:
            h.wait()

        @pl.when(i < HALF - 1)
        def _():
            pltpu.async_copy(
                edge_hbm.at[1, pl.ds((b_a + 2) * CH, CH)], didx_a, isem_a
            )

        for h in hs_b:
            h.wait()

        @pl.when(i < HALF - 1)
        def _():
            pltpu.async_copy(
                edge_hbm.at[1, pl.ds((b_b + 2) * CH, CH)], didx_b, isem_b
            )

        return carry

    lax.fori_loop(0, HALF, body, 0)

    plsc.subcore_barrier()
    pltpu.sync_copy(
        acc_sh.at[pl.ds(sid * OUT_SLICE, OUT_SLICE)],
        out_hbm.at[cid, pl.ds(sid * OUT_SLICE, OUT_SLICE)],
    )


@functools.partial(
    pl.kernel,
    out_type=jax.ShapeDtypeStruct((NUM_CORES, NPAD), jnp.float32),
    mesh=_mesh(),
    scratch_types=[
        pltpu.VMEM((NPAD,), jnp.float32),
        pltpu.VMEM((CH, LANE), jnp.int32),
        pltpu.VMEM((CH, LANE), jnp.int32),
        pltpu.VMEM((CH, LANE), jnp.int32),
        pltpu.VMEM((CH, LANE), jnp.int32),
        pltpu.VMEM((CH, LANE), jnp.float32),
        pltpu.VMEM((CH, LANE), jnp.float32),
        pltpu.VMEM_SHARED((NPAD,), jnp.float32),
        pltpu.SemaphoreType.DMA,
        pltpu.SemaphoreType.DMA,
        pltpu.SemaphoreType.DMA,
        pltpu.SemaphoreType.DMA,
    ],
    compiler_params=pltpu.CompilerParams(needs_layout_passes=False),
)
def _sc_seg(edge_hbm, zeros_hbm, tab_hbm, out_hbm, tab_v,
            sidx_a, sidx_b, didx_a, didx_b, val_a, val_b, acc_sh,
            isem_a, isem_b, ssem_a, ssem_b):
    cid = lax.axis_index("c")
    sid = lax.axis_index("s")
    w = sid * NUM_CORES + cid

    @pl.when(sid == 0)
    def _():
        pltpu.sync_copy(zeros_hbm, acc_sh)

    lo = w * BPW
    pltpu.async_copy(edge_hbm.at[0, pl.ds(lo * CH, CH)], sidx_a, isem_a)
    pltpu.async_copy(edge_hbm.at[1, pl.ds(lo * CH, CH)], didx_a, isem_a)
    pltpu.async_copy(edge_hbm.at[0, pl.ds((lo + 1) * CH, CH)], sidx_b, isem_b)
    pltpu.async_copy(edge_hbm.at[1, pl.ds((lo + 1) * CH, CH)], didx_b, isem_b)

    pltpu.sync_copy(tab_hbm, tab_v)
    plsc.subcore_barrier()

    def gather_rows(sidx_v, val_v):
        for j in range(CH):
            for i in range(LANE // 16):
                idx16 = sidx_v[j, pl.ds(i * 16, 16)]
                val_v[j, pl.ds(i * 16, 16)] = plsc.load_gather(tab_v, [idx16])

    def body(i, carry):
        b_a = lo + 2 * i
        b_b = b_a + 1
        pltpu.make_async_copy(
            edge_hbm.at[0, pl.ds(b_a * CH, CH)], sidx_a, isem_a
        ).wait()
        pltpu.make_async_copy(
            edge_hbm.at[1, pl.ds(b_a * CH, CH)], didx_a, isem_a
        ).wait()
        gather_rows(sidx_a, val_a)
        hs_a = [
            pltpu.async_copy(val_a.at[j], acc_sh.at[didx_a.at[j]], ssem_a, add=True)
            for j in range(CH)
        ]
        pltpu.make_async_copy(
            edge_hbm.at[0, pl.ds(b_b * CH, CH)], sidx_b, isem_b
        ).wait()
        pltpu.make_async_copy(
            edge_hbm.at[1, pl.ds(b_b * CH, CH)], didx_b, isem_b
        ).wait()
        gather_rows(sidx_b, val_b)
        hs_b = [
            pltpu.async_copy(val_b.at[j], acc_sh.at[didx_b.at[j]], ssem_b, add=True)
            for j in range(CH)
        ]
        for h in hs_a:
            h.wait()

        @pl.when(i < HALF - 1)
        def _():
            pltpu.async_copy(edge_hbm.at[0, pl.ds((b_a + 2) * CH, CH)], sidx_a, isem_a)
            pltpu.async_copy(edge_hbm.at[1, pl.ds((b_a + 2) * CH, CH)], didx_a, isem_a)

        for h in hs_b:
            h.wait()

        @pl.when(i < HALF - 1)
        def _():
            pltpu.async_copy(edge_hbm.at[0, pl.ds((b_b + 2) * CH, CH)], sidx_b, isem_b)
            pltpu.async_copy(edge_hbm.at[1, pl.ds((b_b + 2) * CH, CH)], didx_b, isem_b)

        return carry

    lax.fori_loop(0, HALF, body, 0)

    plsc.subcore_barrier()
    pltpu.sync_copy(
        acc_sh.at[pl.ds(sid * OUT_SLICE, OUT_SLICE)],
        out_hbm.at[cid, pl.ds(sid * OUT_SLICE, OUT_SLICE)],
    )


def _d1_body(p_ref, x_ref, dinv_ref, u_ref):
    deg = p_ref[0] + p_ref[1] + 1.0
    dinv = lax.rsqrt(deg)
    dinv_ref[...] = dinv
    u_ref[...] = dinv * x_ref[...]


_d1 = pl.pallas_call(
    _d1_body,
    out_shape=(
        jax.ShapeDtypeStruct((RPAD, LANE), jnp.float32),
        jax.ShapeDtypeStruct((RPAD, LANE), jnp.float32),
    ),
)


def _d2_body(q_ref, dinv_ref, u_ref, w1_ref, b1_ref, w2_ref, v_ref):
    dinv = dinv_ref[...]
    s1 = dinv * (q_ref[0] + q_ref[1] + u_ref[...])
    t = jnp.zeros_like(s1)
    for k in range(HIDDEN):
        t = t + jnp.maximum(s1 * w1_ref[0, k] + b1_ref[0, k], 0.0) * w2_ref[0, k]
    v_ref[...] = dinv * t


_d2 = pl.pallas_call(
    _d2_body,
    in_specs=[
        pl.BlockSpec(memory_space=pltpu.VMEM),
        pl.BlockSpec(memory_space=pltpu.VMEM),
        pl.BlockSpec(memory_space=pltpu.VMEM),
        pl.BlockSpec(memory_space=pltpu.SMEM),
        pl.BlockSpec(memory_space=pltpu.SMEM),
        pl.BlockSpec(memory_space=pltpu.SMEM),
    ],
    out_shape=jax.ShapeDtypeStruct((RPAD, LANE), jnp.float32),
)


def _d3_body(r_ref, dinv_ref, v_ref, b2_ref, o_ref):
    o_ref[...] = dinv_ref[...] * (r_ref[0] + r_ref[1] + v_ref[...]) + b2_ref[0]


_d3 = pl.pallas_call(
    _d3_body,
    in_specs=[
        pl.BlockSpec(memory_space=pltpu.VMEM),
        pl.BlockSpec(memory_space=pltpu.VMEM),
        pl.BlockSpec(memory_space=pltpu.VMEM),
        pl.BlockSpec(memory_space=pltpu.SMEM),
    ],
    out_shape=jax.ShapeDtypeStruct((RPAD, LANE), jnp.float32),
)


def kernel(x, edge_index, W1, b1, W2, b2):
    ei = jnp.pad(
        edge_index.astype(jnp.int32),
        ((0, 0), (0, EROWSP * LANE - N_EDGES)),
        constant_values=N_NODES,
    ).reshape(2, EROWSP, LANE)
    xpad = jnp.pad(x[:, 0], (0, NPAD - N_NODES)).reshape(RPAD, LANE)
    zeros = jnp.zeros((NPAD,), jnp.float32)

    p = _sc_count(ei, zeros)
    dinv, u = _d1(p.reshape(NUM_CORES, RPAD, LANE), xpad)

    g1 = _sc_seg(ei, zeros, u.reshape(NPAD))
    v = _d2(
        g1.reshape(NUM_CORES, RPAD, LANE),
        dinv,
        u,
        W1.reshape(1, HIDDEN),
        b1.reshape(1, HIDDEN),
        W2.reshape(1, HIDDEN),
    )

    g2 = _sc_seg(ei, zeros, v.reshape(NPAD))
    out = _d3(g2.reshape(NUM_CORES, RPAD, LANE), dinv, v, b2)

    return out.reshape(NPAD)[:N_NODES].reshape(N_NODES, 1)


# speedup vs baseline: 380.7440x; 1.1914x over previous
import functools

import jax
import jax.numpy as jnp
from jax import lax
from jax.experimental import pallas as pl
from jax.experimental.pallas import tpu as pltpu
from jax.experimental.pallas import tpu_sc as plsc

N_NODES = 100000
HIDDEN = 16
N_EDGES = 3200000
LANE = 128
NPAD = 100352
RPAD = NPAD // LANE
NUM_CORES = 2
NUM_SUBCORES = 16
NUM_WORKERS = NUM_CORES * NUM_SUBCORES
OUT_SLICE = NPAD // NUM_SUBCORES

CH = 8
BV = CH * LANE
NBLK = N_EDGES // BV
BLK_BASE = NBLK // NUM_WORKERS
BLK_EXTRA = NBLK % NUM_WORKERS
FULL_ITERS = (BLK_BASE - 1) // 2


def _mesh():
    return plsc.VectorSubcoreMesh(core_axis_name="c", subcore_axis_name="s")


def _worker_blocks(w):
    lo = w * BLK_BASE + jnp.minimum(w, BLK_EXTRA)
    cnt = BLK_BASE + jnp.where(w < BLK_EXTRA, 1, 0)
    return lo, cnt


@functools.partial(
    pl.kernel,
    out_type=jax.ShapeDtypeStruct((NUM_CORES, NPAD), jnp.float32),
    mesh=_mesh(),
    scratch_types=[
        pltpu.VMEM((BV,), jnp.int32),
        pltpu.VMEM((BV,), jnp.int32),
        pltpu.VMEM((LANE,), jnp.float32),
        pltpu.VMEM_SHARED((NPAD,), jnp.float32),
        pltpu.SemaphoreType.DMA,
        pltpu.SemaphoreType.DMA,
        pltpu.SemaphoreType.DMA,
        pltpu.SemaphoreType.DMA,
    ],
)
def _sc_count(edge_hbm, zeros_hbm, out_hbm, didx_a, didx_b, ones_v, acc_sh,
              isem_a, isem_b, ssem_a, ssem_b):
    cid = lax.axis_index("c")
    sid = lax.axis_index("s")
    w = sid * NUM_CORES + cid

    for i in range(LANE // 16):
        ones_v[pl.ds(i * 16, 16)] = jnp.full((16,), 1.0, jnp.float32)

    @pl.when(sid == 0)
    def _():
        pltpu.sync_copy(zeros_hbm, acc_sh)

    plsc.subcore_barrier()

    lo, cnt = _worker_blocks(w)

    def fire_idx(b, didx_v, isem):
        pltpu.async_copy(edge_hbm.at[1, pl.ds(b * BV, BV)], didx_v, isem)

    def wait_idx(b, didx_v, isem):
        pltpu.make_async_copy(edge_hbm.at[1, pl.ds(b * BV, BV)], didx_v, isem).wait()

    def fire_slot(didx_v, ssem):
        return [
            pltpu.async_copy(
                ones_v,
                acc_sh.at[didx_v.at[pl.ds(j * LANE, LANE)]],
                ssem,
                add=True,
            )
            for j in range(CH)
        ]

    fire_idx(lo, didx_a, isem_a)
    fire_idx(lo + 1, didx_b, isem_b)

    def body(i, carry):
        b_a = lo + 2 * i
        b_b = b_a + 1
        wait_idx(b_a, didx_a, isem_a)
        hs_a = fire_slot(didx_a, ssem_a)
        wait_idx(b_b, didx_b, isem_b)
        hs_b = fire_slot(didx_b, ssem_b)
        for h in hs_a:
            h.wait()
        fire_idx(b_a + 2, didx_a, isem_a)
        for h in hs_b:
            h.wait()

        @pl.when(2 * i + 3 < cnt)
        def _():
            fire_idx(b_b + 2, didx_b, isem_b)

        return carry

    lax.fori_loop(0, FULL_ITERS, body, 0)

    b_a = lo + 2 * FULL_ITERS
    wait_idx(b_a, didx_a, isem_a)
    hs_a = fire_slot(didx_a, ssem_a)

    @pl.when(2 * FULL_ITERS + 1 < cnt)
    def _():
        wait_idx(b_a + 1, didx_b, isem_b)
        hs_b = fire_slot(didx_b, ssem_b)
        for h in hs_b:
            h.wait()

    for h in hs_a:
        h.wait()

    plsc.subcore_barrier()
    pltpu.sync_copy(
        acc_sh.at[pl.ds(sid * OUT_SLICE, OUT_SLICE)],
        out_hbm.at[cid, pl.ds(sid * OUT_SLICE, OUT_SLICE)],
    )


@functools.partial(
    pl.kernel,
    out_type=jax.ShapeDtypeStruct((NUM_CORES, NPAD), jnp.float32),
    mesh=_mesh(),
    scratch_types=[
        pltpu.VMEM((NPAD,), jnp.float32),
        pltpu.VMEM((BV,), jnp.int32),
        pltpu.VMEM((BV,), jnp.int32),
        pltpu.VMEM((BV,), jnp.int32),
        pltpu.VMEM((BV,), jnp.int32),
        pltpu.VMEM((BV,), jnp.float32),
        pltpu.VMEM((BV,), jnp.float32),
        pltpu.VMEM_SHARED((NPAD,), jnp.float32),
        pltpu.SemaphoreType.DMA,
        pltpu.SemaphoreType.DMA,
        pltpu.SemaphoreType.DMA,
        pltpu.SemaphoreType.DMA,
    ],
    compiler_params=pltpu.CompilerParams(needs_layout_passes=False),
)
def _sc_seg(edge_hbm, zeros_hbm, tab_hbm, out_hbm, tab_v,
            sidx_a, sidx_b, didx_a, didx_b, val_a, val_b, acc_sh,
            isem_a, isem_b, ssem_a, ssem_b):
    cid = lax.axis_index("c")
    sid = lax.axis_index("s")
    w = sid * NUM_CORES + cid

    @pl.when(sid == 0)
    def _():
        pltpu.sync_copy(zeros_hbm, acc_sh)

    lo, cnt = _worker_blocks(w)

    def fire_idx(b, sidx_v, didx_v, isem):
        pltpu.async_copy(edge_hbm.at[0, pl.ds(b * BV, BV)], sidx_v, isem)
        pltpu.async_copy(edge_hbm.at[1, pl.ds(b * BV, BV)], didx_v, isem)

    def wait_idx(b, sidx_v, didx_v, isem):
        pltpu.make_async_copy(edge_hbm.at[0, pl.ds(b * BV, BV)], sidx_v, isem).wait()
        pltpu.make_async_copy(edge_hbm.at[1, pl.ds(b * BV, BV)], didx_v, isem).wait()

    fire_idx(lo, sidx_a, didx_a, isem_a)
    fire_idx(lo + 1, sidx_b, didx_b, isem_b)

    pltpu.sync_copy(tab_hbm, tab_v)
    plsc.subcore_barrier()

    def gather_fire_slot(sidx_v, didx_v, val_v, ssem):
        for k in range(BV // 16):
            idx16 = sidx_v[pl.ds(k * 16, 16)]
            val_v[pl.ds(k * 16, 16)] = plsc.load_gather(tab_v, [idx16])
        return [
            pltpu.async_copy(
                val_v.at[pl.ds(j * LANE, LANE)],
                acc_sh.at[didx_v.at[pl.ds(j * LANE, LANE)]],
                ssem,
                add=True,
            )
            for j in range(CH)
        ]

    def body(i, carry):
        b_a = lo + 2 * i
        b_b = b_a + 1
        wait_idx(b_a, sidx_a, didx_a, isem_a)
        hs_a = gather_fire_slot(sidx_a, didx_a, val_a, ssem_a)
        wait_idx(b_b, sidx_b, didx_b, isem_b)
        hs_b = gather_fire_slot(sidx_b, didx_b, val_b, ssem_b)
        for h in hs_a:
            h.wait()
        fire_idx(b_a + 2, sidx_a, didx_a, isem_a)
        for h in hs_b:
            h.wait()

        @pl.when(2 * i + 3 < cnt)
        def _():
            fire_idx(b_b + 2, sidx_b, didx_b, isem_b)

        return carry

    lax.fori_loop(0, FULL_ITERS, body, 0)

    b_a = lo + 2 * FULL_ITERS
    wait_idx(b_a, sidx_a, didx_a, isem_a)
    hs_a = gather_fire_slot(sidx_a, didx_a, val_a, ssem_a)

    @pl.when(2 * FULL_ITERS + 1 < cnt)
    def _():
        wait_idx(b_a + 1, sidx_b, didx_b, isem_b)
        hs_b = gather_fire_slot(sidx_b, didx_b, val_b, ssem_b)
        for h in hs_b:
            h.wait()

    for h in hs_a:
        h.wait()

    plsc.subcore_barrier()
    pltpu.sync_copy(
        acc_sh.at[pl.ds(sid * OUT_SLICE, OUT_SLICE)],
        out_hbm.at[cid, pl.ds(sid * OUT_SLICE, OUT_SLICE)],
    )


def _d1_body(p_ref, x_ref, dinv_ref, u_ref):
    deg = p_ref[0] + p_ref[1] + 1.0
    dinv = lax.rsqrt(deg)
    dinv_ref[...] = dinv
    u_ref[...] = dinv * x_ref[...]


_d1 = pl.pallas_call(
    _d1_body,
    out_shape=(
        jax.ShapeDtypeStruct((RPAD, LANE), jnp.float32),
        jax.ShapeDtypeStruct((RPAD, LANE), jnp.float32),
    ),
)


def _d2_body(q_ref, dinv_ref, u_ref, w1_ref, b1_ref, w2_ref, v_ref):
    dinv = dinv_ref[...]
    s1 = dinv * (q_ref[0] + q_ref[1] + u_ref[...])
    t = jnp.zeros_like(s1)
    for k in range(HIDDEN):
        t = t + jnp.maximum(s1 * w1_ref[0, k] + b1_ref[0, k], 0.0) * w2_ref[0, k]
    v_ref[...] = dinv * t


_d2 = pl.pallas_call(
    _d2_body,
    in_specs=[
        pl.BlockSpec(memory_space=pltpu.VMEM),
        pl.BlockSpec(memory_space=pltpu.VMEM),
        pl.BlockSpec(memory_space=pltpu.VMEM),
        pl.BlockSpec(memory_space=pltpu.SMEM),
        pl.BlockSpec(memory_space=pltpu.SMEM),
        pl.BlockSpec(memory_space=pltpu.SMEM),
    ],
    out_shape=jax.ShapeDtypeStruct((RPAD, LANE), jnp.float32),
)


def _d3_body(r_ref, dinv_ref, v_ref, b2_ref, o_ref):
    o_ref[...] = dinv_ref[...] * (r_ref[0] + r_ref[1] + v_ref[...]) + b2_ref[0]


_d3 = pl.pallas_call(
    _d3_body,
    in_specs=[
        pl.BlockSpec(memory_space=pltpu.VMEM),
        pl.BlockSpec(memory_space=pltpu.VMEM),
        pl.BlockSpec(memory_space=pltpu.VMEM),
        pl.BlockSpec(memory_space=pltpu.SMEM),
    ],
    out_shape=jax.ShapeDtypeStruct((RPAD, LANE), jnp.float32),
)


def kernel(x, edge_index, W1, b1, W2, b2):
    ei = edge_index.astype(jnp.int32)
    xpad = jnp.pad(x[:, 0], (0, NPAD - N_NODES)).reshape(RPAD, LANE)
    zeros = jnp.zeros((NPAD,), jnp.float32)

    p = _sc_count(ei, zeros)
    dinv, u = _d1(p.reshape(NUM_CORES, RPAD, LANE), xpad)

    g1 = _sc_seg(ei, zeros, u.reshape(NPAD))
    v = _d2(
        g1.reshape(NUM_CORES, RPAD, LANE),
        dinv,
        u,
        W1.reshape(1, HIDDEN),
        b1.reshape(1, HIDDEN),
        W2.reshape(1, HIDDEN),
    )

    g2 = _sc_seg(ei, zeros, v.reshape(NPAD))
    out = _d3(g2.reshape(NUM_CORES, RPAD, LANE), dinv, v, b2)

    return out.reshape(NPAD)[:N_NODES].reshape(N_NODES, 1)
